# baseline (speedup 1.0000x reference)
import functools

import jax
import jax.numpy as jnp
from jax import lax
from jax.experimental import pallas as pl
from jax.experimental.pallas import tpu as pltpu
from jax.experimental.pallas import tpu_sc as plsc

N_NODES = 10000
N_EDGES = 320000
IN_CH = 128
HIDDEN = 128

NC = 2
NS = 16
NW = NC * NS

NPAD_DEG = 10240
DEG_PER_TILE = NPAD_DEG // NS
NPAD = 10000
ROWS_PER_TILE = NPAD // NS
CHUNK = 128
SUP = 4
SUP_DEG = 8
CH_PER_W = 80
NSUP = CH_PER_W // SUP
NSUP_DEG = CH_PER_W // SUP_DEG
EPW = CH_PER_W * CHUNK
EPAD = EPW * NW
NCHROWS = EPAD // CHUNK
NBUF = 3

_mesh = plsc.VectorSubcoreMesh(core_axis_name="c", subcore_axis_name="s")


@functools.partial(
    pl.kernel,
    mesh=_mesh,
    out_type=jax.ShapeDtypeStruct((NC, NPAD_DEG), jnp.float32),
    scratch_types=[
        pltpu.VMEM_SHARED((NPAD_DEG,), jnp.float32),
        pltpu.VMEM((SUP_DEG, CHUNK), jnp.int32),
        pltpu.VMEM((SUP_DEG, CHUNK), jnp.float32),
        pltpu.VMEM((DEG_PER_TILE,), jnp.float32),
        pltpu.SemaphoreType.DMA,
        pltpu.SemaphoreType.DMA,
    ],
)
def _deg_kernel(dst_hbm, ew_hbm, out_hbm, accum, didx, ewv, zbuf, sem_i, sem_s):
    c = lax.axis_index("c")
    s = lax.axis_index("s")
    wid = c * NS + s
    wbase = wid * CH_PER_W
    z16 = jnp.zeros((16,), jnp.float32)
    for i in range(DEG_PER_TILE // 16):
        zbuf[pl.ds(i * 16, 16)] = z16
    pltpu.sync_copy(zbuf, accum.at[pl.ds(s * DEG_PER_TILE, DEG_PER_TILE)])
    plsc.subcore_barrier()

    def super_body(sp, carry):
        base = wbase + sp * SUP_DEG
        d1 = pltpu.async_copy(dst_hbm.at[pl.ds(base, SUP_DEG)], didx, sem_i)
        d2 = pltpu.async_copy(ew_hbm.at[pl.ds(base, SUP_DEG)], ewv, sem_i)
        d1.wait()
        d2.wait()
        descs = []
        for k in range(SUP_DEG):
            descs.append(
                pltpu.async_copy(ewv.at[k], accum.at[didx.at[k]], sem_s, add=True))
        for d in descs:
            d.wait()
        return carry

    lax.fori_loop(0, NSUP_DEG, super_body, 0)
    plsc.subcore_barrier()
    pltpu.sync_copy(
        accum.at[pl.ds(s * DEG_PER_TILE, DEG_PER_TILE)],
        out_hbm.at[c, pl.ds(s * DEG_PER_TILE, DEG_PER_TILE)],
    )


def _mm_body(x_ref, w_ref, h_ref):
    h_ref[...] = jnp.dot(x_ref[...], w_ref[...], preferred_element_type=jnp.float32)


def _mm_call(x, W):
    return pl.pallas_call(
        _mm_body,
        out_shape=jax.ShapeDtypeStruct((N_NODES, HIDDEN), jnp.float32),
    )(x, W)


def _scale_body(h_ref, d0_ref, d1_ref, hs_ref, dis_ref):
    deg = d0_ref[pl.ds(0, N_NODES), :] + d1_ref[pl.ds(0, N_NODES), :] + 1.0
    dis = lax.rsqrt(deg)
    hs_ref[...] = h_ref[...] * dis
    dis_ref[...] = dis


def _scale_call(h, d0, d1):
    return pl.pallas_call(
        _scale_body,
        out_shape=[
            jax.ShapeDtypeStruct((N_NODES, HIDDEN), jnp.float32),
            jax.ShapeDtypeStruct((N_NODES, 1), jnp.float32),
        ],
    )(h, d0, d1)


@functools.partial(
    pl.kernel,
    mesh=_mesh,
    out_type=[
        jax.ShapeDtypeStruct((NPAD, HIDDEN), jnp.float32),
        jax.ShapeDtypeStruct((NPAD, HIDDEN), jnp.float32),
    ],
    scratch_types=[
        pltpu.VMEM_SHARED((NPAD, HIDDEN), jnp.float32),
        pltpu.VMEM((SUP, CHUNK), jnp.int32),
        pltpu.VMEM((SUP, CHUNK), jnp.int32),
        pltpu.VMEM((SUP, CHUNK), jnp.float32),
        [pltpu.VMEM((CHUNK, HIDDEN), jnp.float32) for _ in range(NBUF)],
        pltpu.SemaphoreType.DMA,
        [pltpu.SemaphoreType.DMA for _ in range(NBUF)],
        [pltpu.SemaphoreType.DMA for _ in range(NBUF)],
    ],
)
def _msg_kernel(hs_hbm, src_hbm, dst_hbm, ew_hbm, out0, out1,
                accum, sidx, didx, ewv, rows, sem_i, sem_g, sem_s):
    c = lax.axis_index("c")
    s = lax.axis_index("s")
    wid = c * NS + s
    wbase = wid * CH_PER_W
    z16 = jnp.zeros((16,), jnp.float32)
    zbuf = rows[0]

    def zero_row(i, carry):
        for q in range(HIDDEN // 16):
            zbuf[i, pl.ds(q * 16, 16)] = z16
        return carry

    @pl.when((c == 0) & (s < NS - 1))
    def _():
        pltpu.sync_copy(hs_hbm.at[pl.ds(s * 640, 640)],
                        accum.at[pl.ds(s * 640, 640)])

    @pl.when((c == 0) & (s == NS - 1))
    def _():
        pltpu.sync_copy(hs_hbm.at[pl.ds(9600, 400)],
                        accum.at[pl.ds(9600, 400)])

    @pl.when(c == 1)
    def _():
        lax.fori_loop(0, CHUNK, zero_row, 0)

    @pl.when((c == 1) & (s < NS - 1))
    def _():
        for t in range(640 // CHUNK):
            pltpu.sync_copy(zbuf, accum.at[pl.ds(s * 640 + t * CHUNK, CHUNK)])

    @pl.when((c == 1) & (s == NS - 1))
    def _():
        for t in range(3):
            pltpu.sync_copy(zbuf, accum.at[pl.ds(9600 + t * CHUNK, CHUNK)])
        pltpu.sync_copy(zbuf.at[pl.ds(0, 16)], accum.at[pl.ds(9984, 16)])

    plsc.subcore_barrier()

    def super_body(sp, carry):
        base = wbase + sp * SUP
        d1 = pltpu.async_copy(src_hbm.at[pl.ds(base, SUP)], sidx, sem_i)
        d2 = pltpu.async_copy(dst_hbm.at[pl.ds(base, SUP)], didx, sem_i)
        d3 = pltpu.async_copy(ew_hbm.at[pl.ds(base, SUP)], ewv, sem_i)
        d1.wait()
        d2.wait()
        d3.wait()
        gd = {}
        sd = {}
        for k in range(NBUF - 1):
            gd[k] = pltpu.async_copy(hs_hbm.at[sidx.at[k]], rows[k], sem_g[k])
        for k in range(SUP):
            j = k % NBUF
            if k == 0:
                gd[NBUF - 1] = pltpu.async_copy(
                    hs_hbm.at[sidx.at[NBUF - 1]], rows[NBUF - 1], sem_g[NBUF - 1])
            elif k + NBUF - 1 < SUP:
                kk = k + NBUF - 1
                sd[kk - NBUF].wait()
                gd[kk] = pltpu.async_copy(
                    hs_hbm.at[sidx.at[kk]], rows[kk % NBUF], sem_g[kk % NBUF])
            gd[k].wait()
            rbuf = rows[j]

            def scale_grp(t, inner, _k=k, _rbuf=rbuf):
                wv = ewv[_k, pl.ds(t * 16, 16)]
                for e in range(16):
                    w = wv[e]
                    jrow = t * 16 + e
                    for q in range(HIDDEN // 16):
                        sl = pl.ds(q * 16, 16)
                        _rbuf[jrow, sl] = _rbuf[jrow, sl] * w
                return inner

            lax.fori_loop(0, CHUNK // 16, scale_grp, 0)
            sd[k] = pltpu.async_copy(rbuf, accum.at[didx.at[k]], sem_s[j], add=True)
        for k in range(SUP - NBUF, SUP):
            sd[k].wait()
        return carry

    lax.fori_loop(0, NSUP, super_body, 0)
    plsc.subcore_barrier()

    @pl.when((c == 0) & (s < NS - 1))
    def _():
        pltpu.sync_copy(accum.at[pl.ds(s * 640, 640)], out0.at[pl.ds(s * 640, 640)])

    @pl.when((c == 0) & (s == NS - 1))
    def _():
        pltpu.sync_copy(accum.at[pl.ds(9600, 400)], out0.at[pl.ds(9600, 400)])

    @pl.when((c == 1) & (s < NS - 1))
    def _():
        pltpu.sync_copy(accum.at[pl.ds(s * 640, 640)], out1.at[pl.ds(s * 640, 640)])

    @pl.when((c == 1) & (s == NS - 1))
    def _():
        pltpu.sync_copy(accum.at[pl.ds(9600, 400)], out1.at[pl.ds(9600, 400)])


_BLK = 400
_NBLK = N_NODES // _BLK


def _out_body(p0_ref, p1_ref, dis_ref, b_ref, o_ref):
    acc = p0_ref[...] + p1_ref[...]
    o_ref[...] = jnp.maximum(acc * dis_ref[...] + b_ref[...], 0.0)


def _out_call(p0, p1, dis, b2):
    return pl.pallas_call(
        _out_body,
        grid=(_NBLK,),
        in_specs=[
            pl.BlockSpec((_BLK, HIDDEN), lambda i: (i, 0)),
            pl.BlockSpec((_BLK, HIDDEN), lambda i: (i, 0)),
            pl.BlockSpec((_BLK, 1), lambda i: (i, 0)),
            pl.BlockSpec((1, HIDDEN), lambda i: (0, 0)),
        ],
        out_specs=pl.BlockSpec((_BLK, HIDDEN), lambda i: (i, 0)),
        out_shape=jax.ShapeDtypeStruct((N_NODES, HIDDEN), jnp.float32),
    )(p0, p1, dis, b2)


def kernel(x, edge_index, edge_weight, W, b):
    src = edge_index[0].astype(jnp.int32)
    dst = edge_index[1].astype(jnp.int32)
    ew = edge_weight.astype(jnp.float32)
    pad = EPAD - N_EDGES
    pidx = jnp.arange(pad, dtype=jnp.int32) % N_NODES
    src = jnp.concatenate([src, pidx]).reshape(NCHROWS, CHUNK)
    dst = jnp.concatenate([dst, pidx]).reshape(NCHROWS, CHUNK)
    ew = jnp.concatenate([ew, jnp.zeros((pad,), jnp.float32)]).reshape(NCHROWS, CHUNK)

    pdeg = _deg_kernel(dst, ew)
    h = _mm_call(x, W)
    d0 = pdeg[0][:, None]
    d1 = pdeg[1][:, None]
    hs, dis = _scale_call(h, d0, d1)
    p0, p1 = _msg_kernel(hs, src, dst, ew)
    out = _out_call(p0, p1, dis, b.reshape(1, HIDDEN))
    return out


# speedup vs baseline: 1.0040x; 1.0040x over previous
import functools

import jax
import jax.numpy as jnp
from jax import lax
from jax.experimental import pallas as pl
from jax.experimental.pallas import tpu as pltpu
from jax.experimental.pallas import tpu_sc as plsc

N_NODES = 10000
N_EDGES = 320000
IN_CH = 128
HIDDEN = 128

NC = 2
NS = 16
NW = NC * NS

NPAD_DEG = 10240
DEG_PER_TILE = NPAD_DEG // NS
NPAD = 10000
ROWS_PER_TILE = NPAD // NS
CHUNK = 128
SUP = 4
SUP_DEG = 8
CH_PER_W = 80
NSUP = CH_PER_W // SUP
NSUP_DEG = CH_PER_W // SUP_DEG
EPW = CH_PER_W * CHUNK
EPAD = EPW * NW
NCHROWS = EPAD // CHUNK
NBUF = 3

_mesh = plsc.VectorSubcoreMesh(core_axis_name="c", subcore_axis_name="s")


@functools.partial(
    pl.kernel,
    mesh=_mesh,
    out_type=jax.ShapeDtypeStruct((NC, NPAD_DEG), jnp.float32),
    scratch_types=[
        pltpu.VMEM_SHARED((NPAD_DEG,), jnp.float32),
        pltpu.VMEM((SUP_DEG, CHUNK), jnp.int32),
        pltpu.VMEM((SUP_DEG, CHUNK), jnp.float32),
        pltpu.VMEM((DEG_PER_TILE,), jnp.float32),
        pltpu.SemaphoreType.DMA,
        pltpu.SemaphoreType.DMA,
    ],
)
def _deg_kernel(dst_hbm, ew_hbm, out_hbm, accum, didx, ewv, zbuf, sem_i, sem_s):
    c = lax.axis_index("c")
    s = lax.axis_index("s")
    wid = c * NS + s
    wbase = wid * CH_PER_W
    z16 = jnp.zeros((16,), jnp.float32)
    for i in range(DEG_PER_TILE // 16):
        zbuf[pl.ds(i * 16, 16)] = z16
    pltpu.sync_copy(zbuf, accum.at[pl.ds(s * DEG_PER_TILE, DEG_PER_TILE)])
    plsc.subcore_barrier()

    def super_body(sp, carry):
        base = wbase + sp * SUP_DEG
        d1 = pltpu.async_copy(dst_hbm.at[pl.ds(base, SUP_DEG)], didx, sem_i)
        d2 = pltpu.async_copy(ew_hbm.at[pl.ds(base, SUP_DEG)], ewv, sem_i)
        d1.wait()
        d2.wait()
        descs = []
        for k in range(SUP_DEG):
            descs.append(
                pltpu.async_copy(ewv.at[k], accum.at[didx.at[k]], sem_s, add=True))
        for d in descs:
            d.wait()
        return carry

    lax.fori_loop(0, NSUP_DEG, super_body, 0)
    plsc.subcore_barrier()
    pltpu.sync_copy(
        accum.at[pl.ds(s * DEG_PER_TILE, DEG_PER_TILE)],
        out_hbm.at[c, pl.ds(s * DEG_PER_TILE, DEG_PER_TILE)],
    )


def _mm_body(x_ref, w_ref, d0_ref, d1_ref, hs_ref, dis_ref):
    deg = d0_ref[pl.ds(0, N_NODES), :] + d1_ref[pl.ds(0, N_NODES), :] + 1.0
    dis = lax.rsqrt(deg)
    h = jnp.dot(x_ref[...], w_ref[...], preferred_element_type=jnp.float32)
    hs_ref[...] = h * dis
    dis_ref[...] = dis


def _mm_call(x, W, d0, d1):
    return pl.pallas_call(
        _mm_body,
        out_shape=[
            jax.ShapeDtypeStruct((N_NODES, HIDDEN), jnp.float32),
            jax.ShapeDtypeStruct((N_NODES, 1), jnp.float32),
        ],
    )(x, W, d0, d1)


@functools.partial(
    pl.kernel,
    mesh=_mesh,
    out_type=[
        jax.ShapeDtypeStruct((NPAD, HIDDEN), jnp.float32),
        jax.ShapeDtypeStruct((NPAD, HIDDEN), jnp.float32),
    ],
    scratch_types=[
        pltpu.VMEM_SHARED((NPAD, HIDDEN), jnp.float32),
        pltpu.VMEM((SUP, CHUNK), jnp.int32),
        pltpu.VMEM((SUP, CHUNK), jnp.int32),
        pltpu.VMEM((SUP, CHUNK), jnp.float32),
        [pltpu.VMEM((CHUNK, HIDDEN), jnp.float32) for _ in range(NBUF)],
        pltpu.SemaphoreType.DMA,
        [pltpu.SemaphoreType.DMA for _ in range(NBUF)],
        [pltpu.SemaphoreType.DMA for _ in range(NBUF)],
    ],
)
def _msg_kernel(hs_hbm, src_hbm, dst_hbm, ew_hbm, out0, out1,
                accum, sidx, didx, ewv, rows, sem_i, sem_g, sem_s):
    c = lax.axis_index("c")
    s = lax.axis_index("s")
    wid = c * NS + s
    wbase = wid * CH_PER_W
    z16 = jnp.zeros((16,), jnp.float32)
    zbuf = rows[0]

    def zero_row(i, carry):
        for q in range(HIDDEN // 16):
            zbuf[i, pl.ds(q * 16, 16)] = z16
        return carry

    @pl.when((c == 0) & (s < NS - 1))
    def _():
        pltpu.sync_copy(hs_hbm.at[pl.ds(s * 640, 640)],
                        accum.at[pl.ds(s * 640, 640)])

    @pl.when((c == 0) & (s == NS - 1))
    def _():
        pltpu.sync_copy(hs_hbm.at[pl.ds(9600, 400)],
                        accum.at[pl.ds(9600, 400)])

    @pl.when(c == 1)
    def _():
        lax.fori_loop(0, CHUNK, zero_row, 0)

    @pl.when((c == 1) & (s < NS - 1))
    def _():
        for t in range(640 // CHUNK):
            pltpu.sync_copy(zbuf, accum.at[pl.ds(s * 640 + t * CHUNK, CHUNK)])

    @pl.when((c == 1) & (s == NS - 1))
    def _():
        for t in range(3):
            pltpu.sync_copy(zbuf, accum.at[pl.ds(9600 + t * CHUNK, CHUNK)])
        pltpu.sync_copy(zbuf.at[pl.ds(0, 16)], accum.at[pl.ds(9984, 16)])

    plsc.subcore_barrier()

    def super_body(sp, carry):
        base = wbase + sp * SUP
        d1 = pltpu.async_copy(src_hbm.at[pl.ds(base, SUP)], sidx, sem_i)
        d2 = pltpu.async_copy(dst_hbm.at[pl.ds(base, SUP)], didx, sem_i)
        d3 = pltpu.async_copy(ew_hbm.at[pl.ds(base, SUP)], ewv, sem_i)
        d1.wait()
        d2.wait()
        d3.wait()
        gd = {}
        sd = {}
        for k in range(NBUF - 1):
            gd[k] = pltpu.async_copy(hs_hbm.at[sidx.at[k]], rows[k], sem_g[k])
        for k in range(SUP):
            j = k % NBUF
            if k == 0:
                gd[NBUF - 1] = pltpu.async_copy(
                    hs_hbm.at[sidx.at[NBUF - 1]], rows[NBUF - 1], sem_g[NBUF - 1])
            elif k + NBUF - 1 < SUP:
                kk = k + NBUF - 1
                sd[kk - NBUF].wait()
                gd[kk] = pltpu.async_copy(
                    hs_hbm.at[sidx.at[kk]], rows[kk % NBUF], sem_g[kk % NBUF])
            gd[k].wait()
            rbuf = rows[j]

            def scale_grp(t, inner, _k=k, _rbuf=rbuf):
                wv = ewv[_k, pl.ds(t * 16, 16)]
                for e in range(16):
                    w = wv[e]
                    jrow = t * 16 + e
                    for q in range(HIDDEN // 16):
                        sl = pl.ds(q * 16, 16)
                        _rbuf[jrow, sl] = _rbuf[jrow, sl] * w
                return inner

            lax.fori_loop(0, CHUNK // 16, scale_grp, 0)
            sd[k] = pltpu.async_copy(rbuf, accum.at[didx.at[k]], sem_s[j], add=True)
        for k in range(SUP - NBUF, SUP):
            sd[k].wait()
        return carry

    lax.fori_loop(0, NSUP, super_body, 0)
    plsc.subcore_barrier()

    @pl.when((c == 0) & (s < NS - 1))
    def _():
        pltpu.sync_copy(accum.at[pl.ds(s * 640, 640)], out0.at[pl.ds(s * 640, 640)])

    @pl.when((c == 0) & (s == NS - 1))
    def _():
        pltpu.sync_copy(accum.at[pl.ds(9600, 400)], out0.at[pl.ds(9600, 400)])

    @pl.when((c == 1) & (s < NS - 1))
    def _():
        pltpu.sync_copy(accum.at[pl.ds(s * 640, 640)], out1.at[pl.ds(s * 640, 640)])

    @pl.when((c == 1) & (s == NS - 1))
    def _():
        pltpu.sync_copy(accum.at[pl.ds(9600, 400)], out1.at[pl.ds(9600, 400)])


_BLK = 400
_NBLK = N_NODES // _BLK


def _out_body(p0_ref, p1_ref, dis_ref, b_ref, o_ref):
    acc = p0_ref[...] + p1_ref[...]
    o_ref[...] = jnp.maximum(acc * dis_ref[...] + b_ref[...], 0.0)


def _out_call(p0, p1, dis, b2):
    return pl.pallas_call(
        _out_body,
        grid=(_NBLK,),
        in_specs=[
            pl.BlockSpec((_BLK, HIDDEN), lambda i: (i, 0)),
            pl.BlockSpec((_BLK, HIDDEN), lambda i: (i, 0)),
            pl.BlockSpec((_BLK, 1), lambda i: (i, 0)),
            pl.BlockSpec((1, HIDDEN), lambda i: (0, 0)),
        ],
        out_specs=pl.BlockSpec((_BLK, HIDDEN), lambda i: (i, 0)),
        out_shape=jax.ShapeDtypeStruct((N_NODES, HIDDEN), jnp.float32),
    )(p0, p1, dis, b2)


def kernel(x, edge_index, edge_weight, W, b):
    src = edge_index[0].astype(jnp.int32)
    dst = edge_index[1].astype(jnp.int32)
    ew = edge_weight.astype(jnp.float32)
    pad = EPAD - N_EDGES
    pidx = jnp.arange(pad, dtype=jnp.int32) % N_NODES
    src = jnp.concatenate([src, pidx]).reshape(NCHROWS, CHUNK)
    dst = jnp.concatenate([dst, pidx]).reshape(NCHROWS, CHUNK)
    ew = jnp.concatenate([ew, jnp.zeros((pad,), jnp.float32)]).reshape(NCHROWS, CHUNK)

    pdeg = _deg_kernel(dst, ew)
    d0 = pdeg[0][:, None]
    d1 = pdeg[1][:, None]
    hs, dis = _mm_call(x, W, d0, d1)
    p0, p1 = _msg_kernel(hs, src, dst, ew)
    out = _out_call(p0, p1, dis, b.reshape(1, HIDDEN))
    return out


# speedup vs baseline: 1.0109x; 1.0069x over previous
import functools

import jax
import jax.numpy as jnp
from jax import lax
from jax.experimental import pallas as pl
from jax.experimental.pallas import tpu as pltpu
from jax.experimental.pallas import tpu_sc as plsc

N_NODES = 10000
N_EDGES = 320000
IN_CH = 128
HIDDEN = 128

NC = 2
NS = 16
NW = NC * NS

NPAD_DEG = 10240
DEG_PER_TILE = NPAD_DEG // NS
NPAD = 10000
ROWS_PER_TILE = NPAD // NS
CHUNK = 128
SUP = 4
SUP_DEG = 8
CH_PER_W = 80
NSUP = CH_PER_W // SUP
NSUP_DEG = CH_PER_W // SUP_DEG
EPW = CH_PER_W * CHUNK
EPAD = EPW * NW
NCHROWS = EPAD // CHUNK
NBUF = 3

_mesh = plsc.VectorSubcoreMesh(core_axis_name="c", subcore_axis_name="s")


@functools.partial(
    pl.kernel,
    mesh=_mesh,
    out_type=jax.ShapeDtypeStruct((NC, NPAD_DEG), jnp.float32),
    scratch_types=[
        pltpu.VMEM_SHARED((NPAD_DEG,), jnp.float32),
        pltpu.VMEM((SUP_DEG, CHUNK), jnp.int32),
        pltpu.VMEM((SUP_DEG, CHUNK), jnp.float32),
        pltpu.VMEM((DEG_PER_TILE,), jnp.float32),
        pltpu.SemaphoreType.DMA,
        pltpu.SemaphoreType.DMA,
    ],
)
def _deg_kernel(dst_hbm, ew_hbm, out_hbm, accum, didx, ewv, zbuf, sem_i, sem_s):
    c = lax.axis_index("c")
    s = lax.axis_index("s")
    wid = c * NS + s
    wbase = wid * CH_PER_W
    z16 = jnp.zeros((16,), jnp.float32)
    for i in range(DEG_PER_TILE // 16):
        zbuf[pl.ds(i * 16, 16)] = z16
    pltpu.sync_copy(zbuf, accum.at[pl.ds(s * DEG_PER_TILE, DEG_PER_TILE)])
    plsc.subcore_barrier()

    def super_body(sp, carry):
        base = wbase + sp * SUP_DEG
        d1 = pltpu.async_copy(dst_hbm.at[pl.ds(base, SUP_DEG)], didx, sem_i)
        d2 = pltpu.async_copy(ew_hbm.at[pl.ds(base, SUP_DEG)], ewv, sem_i)
        d1.wait()
        d2.wait()
        descs = []
        for k in range(SUP_DEG):
            descs.append(
                pltpu.async_copy(ewv.at[k], accum.at[didx.at[k]], sem_s, add=True))
        for d in descs:
            d.wait()
        return carry

    lax.fori_loop(0, NSUP_DEG, super_body, 0)
    plsc.subcore_barrier()
    pltpu.sync_copy(
        accum.at[pl.ds(s * DEG_PER_TILE, DEG_PER_TILE)],
        out_hbm.at[c, pl.ds(s * DEG_PER_TILE, DEG_PER_TILE)],
    )


def _mm_body(x_ref, w_ref, d0_ref, d1_ref, hs_ref, dis_ref):
    deg = d0_ref[pl.ds(0, N_NODES), :] + d1_ref[pl.ds(0, N_NODES), :] + 1.0
    dis = lax.rsqrt(deg)
    h = jnp.dot(x_ref[...], w_ref[...], preferred_element_type=jnp.float32)
    hs_ref[...] = h * dis
    dis_ref[...] = dis


def _mm_call(x, W, d0, d1):
    return pl.pallas_call(
        _mm_body,
        out_shape=[
            jax.ShapeDtypeStruct((N_NODES, HIDDEN), jnp.float32),
            jax.ShapeDtypeStruct((N_NODES, 1), jnp.float32),
        ],
    )(x, W, d0, d1)


@functools.partial(
    pl.kernel,
    mesh=_mesh,
    out_type=[
        jax.ShapeDtypeStruct((NPAD, HIDDEN), jnp.float32),
        jax.ShapeDtypeStruct((NPAD, HIDDEN), jnp.float32),
    ],
    scratch_types=[
        pltpu.VMEM_SHARED((NPAD, HIDDEN), jnp.float32),
        pltpu.VMEM((SUP, CHUNK), jnp.int32),
        pltpu.VMEM((SUP, CHUNK), jnp.int32),
        pltpu.VMEM((SUP, CHUNK), jnp.float32),
        [pltpu.VMEM((CHUNK, HIDDEN), jnp.float32) for _ in range(NBUF)],
        pltpu.SemaphoreType.DMA,
        [pltpu.SemaphoreType.DMA for _ in range(NBUF)],
        [pltpu.SemaphoreType.DMA for _ in range(NBUF)],
    ],
)
def _msg_kernel(hs_hbm, src_hbm, dst_hbm, ew_hbm, out0, out1,
                accum, sidx, didx, ewv, rows, sem_i, sem_g, sem_s):
    c = lax.axis_index("c")
    s = lax.axis_index("s")
    wid = c * NS + s
    wbase = wid * CH_PER_W
    z16 = jnp.zeros((16,), jnp.float32)
    zbuf = rows[0]

    def zero_row(i, carry):
        for q in range(HIDDEN // 16):
            zbuf[i, pl.ds(q * 16, 16)] = z16
        return carry

    lax.fori_loop(0, CHUNK, zero_row, 0)

    @pl.when(s < NS - 1)
    def _():
        for t in range(640 // CHUNK):
            pltpu.sync_copy(zbuf, accum.at[pl.ds(s * 640 + t * CHUNK, CHUNK)])

    @pl.when(s == NS - 1)
    def _():
        for t in range(3):
            pltpu.sync_copy(zbuf, accum.at[pl.ds(9600 + t * CHUNK, CHUNK)])
        pltpu.sync_copy(zbuf.at[pl.ds(0, 16)], accum.at[pl.ds(9984, 16)])

    plsc.subcore_barrier()

    def super_body(sp, carry):
        base = wbase + sp * SUP
        d1 = pltpu.async_copy(src_hbm.at[pl.ds(base, SUP)], sidx, sem_i)
        d2 = pltpu.async_copy(dst_hbm.at[pl.ds(base, SUP)], didx, sem_i)
        d3 = pltpu.async_copy(ew_hbm.at[pl.ds(base, SUP)], ewv, sem_i)
        d1.wait()
        d2.wait()
        d3.wait()
        gd = {}
        sd = {}
        for k in range(NBUF - 1):
            gd[k] = pltpu.async_copy(hs_hbm.at[sidx.at[k]], rows[k], sem_g[k])
        for k in range(SUP):
            j = k % NBUF
            if k == 0:
                gd[NBUF - 1] = pltpu.async_copy(
                    hs_hbm.at[sidx.at[NBUF - 1]], rows[NBUF - 1], sem_g[NBUF - 1])
            elif k + NBUF - 1 < SUP:
                kk = k + NBUF - 1
                sd[kk - NBUF].wait()
                gd[kk] = pltpu.async_copy(
                    hs_hbm.at[sidx.at[kk]], rows[kk % NBUF], sem_g[kk % NBUF])
            gd[k].wait()
            rbuf = rows[j]

            def scale_grp(t, inner, _k=k, _rbuf=rbuf):
                wv = ewv[_k, pl.ds(t * 16, 16)]
                for e in range(16):
                    w = wv[e]
                    jrow = t * 16 + e
                    for q in range(HIDDEN // 16):
                        sl = pl.ds(q * 16, 16)
                        _rbuf[jrow, sl] = _rbuf[jrow, sl] * w
                return inner

            lax.fori_loop(0, CHUNK // 16, scale_grp, 0)
            sd[k] = pltpu.async_copy(rbuf, accum.at[didx.at[k]], sem_s[j], add=True)
        for k in range(SUP - NBUF, SUP):
            sd[k].wait()
        return carry

    lax.fori_loop(0, NSUP, super_body, 0)
    plsc.subcore_barrier()

    @pl.when((c == 0) & (s < NS - 1))
    def _():
        pltpu.sync_copy(accum.at[pl.ds(s * 640, 640)], out0.at[pl.ds(s * 640, 640)])

    @pl.when((c == 0) & (s == NS - 1))
    def _():
        pltpu.sync_copy(accum.at[pl.ds(9600, 400)], out0.at[pl.ds(9600, 400)])

    @pl.when((c == 1) & (s < NS - 1))
    def _():
        pltpu.sync_copy(accum.at[pl.ds(s * 640, 640)], out1.at[pl.ds(s * 640, 640)])

    @pl.when((c == 1) & (s == NS - 1))
    def _():
        pltpu.sync_copy(accum.at[pl.ds(9600, 400)], out1.at[pl.ds(9600, 400)])


_BLK = 400
_NBLK = N_NODES // _BLK


def _out_body(p0_ref, p1_ref, hs_ref, dis_ref, b_ref, o_ref):
    acc = p0_ref[...] + p1_ref[...] + hs_ref[...]
    o_ref[...] = jnp.maximum(acc * dis_ref[...] + b_ref[...], 0.0)


def _out_call(p0, p1, hs, dis, b2):
    return pl.pallas_call(
        _out_body,
        grid=(_NBLK,),
        in_specs=[
            pl.BlockSpec((_BLK, HIDDEN), lambda i: (i, 0)),
            pl.BlockSpec((_BLK, HIDDEN), lambda i: (i, 0)),
            pl.BlockSpec((_BLK, HIDDEN), lambda i: (i, 0)),
            pl.BlockSpec((_BLK, 1), lambda i: (i, 0)),
            pl.BlockSpec((1, HIDDEN), lambda i: (0, 0)),
        ],
        out_specs=pl.BlockSpec((_BLK, HIDDEN), lambda i: (i, 0)),
        out_shape=jax.ShapeDtypeStruct((N_NODES, HIDDEN), jnp.float32),
    )(p0, p1, hs, dis, b2)


def kernel(x, edge_index, edge_weight, W, b):
    src = edge_index[0].astype(jnp.int32)
    dst = edge_index[1].astype(jnp.int32)
    ew = edge_weight.astype(jnp.float32)
    pad = EPAD - N_EDGES
    pidx = jnp.arange(pad, dtype=jnp.int32) % N_NODES
    src = jnp.concatenate([src, pidx]).reshape(NCHROWS, CHUNK)
    dst = jnp.concatenate([dst, pidx]).reshape(NCHROWS, CHUNK)
    ew = jnp.concatenate([ew, jnp.zeros((pad,), jnp.float32)]).reshape(NCHROWS, CHUNK)

    pdeg = _deg_kernel(dst, ew)
    d0 = pdeg[0][:, None]
    d1 = pdeg[1][:, None]
    hs, dis = _mm_call(x, W, d0, d1)
    p0, p1 = _msg_kernel(hs, src, dst, ew)
    out = _out_call(p0, p1, hs, dis, b.reshape(1, HIDDEN))
    return out
